# baseline (speedup 1.0000x reference)
import functools

import jax
import jax.numpy as jnp
from jax import lax
from jax.experimental import pallas as pl
from jax.experimental.pallas import tpu as pltpu
from jax.experimental.pallas import tpu_sc as plsc

N = 50000
E = 800000
N_PAD = 51200
STRIPE = N_PAD // 16
NCH = 400
CHUNKS = NCH * 16
CHUNKS_T = CHUNKS + 16
E_PAD = CHUNKS * 128
NODE_BLK = 5000

_SC_MESH = plsc.VectorSubcoreMesh(core_axis_name="c", subcore_axis_name="s")
_PIB = jax.lax.GatherScatterMode.PROMISE_IN_BOUNDS


def _deg_body(col_hbm, ew_hbm, out_hbm, dacc, zero_v, col_v, ew_v, sem):
    c = lax.axis_index("c")
    s = lax.axis_index("s")

    @pl.loop(0, 512 // 16)
    def _(i):
        zero_v[pl.ds(i * 16, 16)] = jnp.zeros((16,), jnp.float32)

    for t in range(STRIPE // 512):
        pltpu.sync_copy(zero_v, dacc.at[pl.ds(s * STRIPE + t * 512, 512)])
    rem = STRIPE % 512
    if rem:
        pltpu.sync_copy(zero_v.at[pl.ds(0, rem)],
                        dacc.at[pl.ds(s * STRIPE + STRIPE - rem, rem)])
    plsc.subcore_barrier()

    w = c * 16 + s

    @pl.loop(0, 50)
    def _(win):
        cb = w * 200 + win * 4
        pltpu.sync_copy(col_hbm.at[pl.ds(cb, 4)], col_v)
        pltpu.sync_copy(ew_hbm.at[pl.ds(cb, 4)], ew_v)
        cps = [
            pltpu.async_copy(ew_v.at[j], dacc.at[col_v.at[j]], sem, add=True)
            for j in range(4)
        ]
        for cp in cps:
            cp.wait()

    plsc.subcore_barrier()
    pltpu.sync_copy(dacc.at[pl.ds(s * STRIPE, STRIPE)],
                    out_hbm.at[c].at[pl.ds(s * STRIPE, STRIPE)])


def _sc_deg(col2d, ew2d):
    kfn = pl.kernel(
        _deg_body,
        out_type=jax.ShapeDtypeStruct((2, N_PAD), jnp.float32),
        mesh=_SC_MESH,
        scratch_types=[
            pltpu.VMEM_SHARED((N_PAD,), jnp.float32),
            pltpu.VMEM((512,), jnp.float32),
            pltpu.VMEM((4, 128), jnp.int32),
            pltpu.VMEM((4, 128), jnp.float32),
            pltpu.SemaphoreType.DMA,
        ],
    )
    return kfn(col2d, ew2d)


def _agg_body(tab_hbm, pidx_hbm, acc_hbm,
              aacc, idx_v, rows_v, gsem, ssem, isem, *, dh, edge_split):
    c = lax.axis_index("c")
    s = lax.axis_index("s")
    nk = dh // 16
    if edge_split:
        nch = CHUNKS // 32
        sbase = (c * 16 + s) * nch
        tref = tab_hbm
    else:
        nch = NCH
        sbase = s * nch
        tref = tab_hbm.at[c]

    def buf(b):
        return rows_v.at[pl.ds(b * 128, 128)]

    @pl.loop(0, 512)
    def _(i):
        for k in range(nk):
            rows_v[i, pl.ds(k * 16, 16)] = jnp.zeros((16,), jnp.float32)

    for t in range(STRIPE // 512):
        pltpu.sync_copy(rows_v,
                        aacc.at[pl.ds(s * STRIPE + t * 512, 512)])
    rem = STRIPE % 512
    if rem:
        pltpu.sync_copy(rows_v.at[pl.ds(0, rem)],
                        aacc.at[pl.ds(s * STRIPE + STRIPE - rem, rem)])
    plsc.subcore_barrier()

    def issue_gather(ip, b, chunk):
        del chunk
        return pltpu.async_copy(
            tref.at[idx_v.at[ip].at[b].at[0]], buf(b), gsem)

    def drain_gather(ip, b):
        pltpu.make_async_copy(
            tref.at[idx_v.at[ip].at[b].at[0]], buf(b), gsem).wait()

    def issue_scatter(ip, b):
        return pltpu.async_copy(
            buf(b), aacc.at[idx_v.at[ip].at[b].at[1]], ssem, add=True)

    def drain_scatter(ip, b):
        pltpu.make_async_copy(
            buf(b), aacc.at[idx_v.at[ip].at[b].at[1]], ssem).wait()

    def issue_idx(ip, chunk):
        return pltpu.async_copy(pidx_hbm.at[pl.ds(chunk, 4)],
                                idx_v.at[ip], isem)

    def drain_idx(ip, chunk):
        pltpu.make_async_copy(pidx_hbm.at[pl.ds(chunk, 4)],
                              idx_v.at[ip], isem).wait()

    def multiply(ip, b):
        @plsc.parallel_loop(0, 128, unroll=8)
        def _(e):
            g16 = (e // 16) * 16
            ewg = plsc.bitcast(idx_v[ip, b, 2, pl.ds(g16, 16)], jnp.float32)
            bc = lax.gather(
                ewg, jnp.broadcast_to((e % 16)[None, None], (16, 1)),
                lax.GatherDimensionNumbers(
                    offset_dims=(), collapsed_slice_dims=(0,),
                    start_index_map=(0,)),
                slice_sizes=(1,), mode=_PIB)
            r = b * 128 + e
            for k in range(nk):
                rows_v[r, pl.ds(k * 16, 16)] = (
                    rows_v[r, pl.ds(k * 16, 16)] * bc)

    def half(ip, cb_mine, cb_next):
        for b in range(4):
            drain_gather(ip, b)
            multiply(ip, b)
            issue_scatter(ip, b)
        drain_idx(1 - ip, cb_next)
        for b in range(4):
            drain_scatter(ip, b)
        for b in range(4):
            issue_gather(1 - ip, b, cb_next)
        issue_idx(ip, cb_next + 4)

    pltpu.sync_copy(pidx_hbm.at[pl.ds(sbase, 4)], idx_v.at[0])
    for b in range(4):
        issue_gather(0, b, sbase + b)
    issue_idx(1, sbase + 4)

    @pl.loop(0, nch // 8)
    def _(k):
        cb = sbase + k * 8
        half(0, cb, cb + 4)
        half(1, cb + 4, cb + 8)

    for b in range(4):
        drain_gather(0, b)
    drain_idx(1, sbase)

    plsc.subcore_barrier()
    pltpu.sync_copy(aacc.at[pl.ds(s * STRIPE, STRIPE)],
                    acc_hbm.at[c].at[pl.ds(s * STRIPE, STRIPE)])


def _sc_agg(tab, pidx, dh, edge_split=False):
    kfn = pl.kernel(
        functools.partial(_agg_body, dh=dh, edge_split=edge_split),
        out_type=jax.ShapeDtypeStruct((2, N_PAD, dh), jnp.float32),
        mesh=_SC_MESH,
        compiler_params=pltpu.CompilerParams(use_tc_tiling_on_sc=False,
                                             needs_layout_passes=False),
        scratch_types=[
            pltpu.VMEM_SHARED((N_PAD, dh), jnp.float32),
            pltpu.VMEM((2, 4, 3, 128), jnp.int32),
            pltpu.VMEM((512, dh), jnp.float32),
            pltpu.SemaphoreType.DMA,
            pltpu.SemaphoreType.DMA,
            pltpu.SemaphoreType.DMA,
        ],
    )
    return kfn(tab, pidx)


def _mm_body(x_ref, w_ref, o_ref):
    x = x_ref[...]
    o_ref[0] = jnp.dot(x, w_ref[0], preferred_element_type=jnp.float32)
    o_ref[1] = jnp.dot(x, w_ref[1], preferred_element_type=jnp.float32)


def _matmul_split(x, w, dh):
    d_in = x.shape[1]
    w2 = jnp.stack([w[:, :dh], w[:, dh:]])
    return pl.pallas_call(
        _mm_body,
        grid=(N // NODE_BLK,),
        in_specs=[
            pl.BlockSpec((NODE_BLK, d_in), lambda i: (i, 0)),
            pl.BlockSpec((2, d_in, dh), lambda i: (0, 0, 0)),
        ],
        out_specs=pl.BlockSpec((2, NODE_BLK, dh), lambda i: (0, i, 0)),
        out_shape=jax.ShapeDtypeStruct((2, N, dh), jnp.float32),
    )(x, w2)


def _scale_body(dis_ref, t_ref, o_ref):
    dis = dis_ref[...]
    o_ref[0] = dis * t_ref[0]
    o_ref[1] = dis * t_ref[1]


def _scale_split(dis, t):
    dh = t.shape[2]
    return pl.pallas_call(
        _scale_body,
        grid=(N // NODE_BLK,),
        in_specs=[
            pl.BlockSpec((NODE_BLK, 1), lambda i: (i, 0)),
            pl.BlockSpec((2, NODE_BLK, dh), lambda i: (0, i, 0)),
        ],
        out_specs=pl.BlockSpec((2, NODE_BLK, dh), lambda i: (0, i, 0)),
        out_shape=jax.ShapeDtypeStruct((2, N, dh), jnp.float32),
    )(dis, t)


def _mid_body(dis_ref, acc_ref, hp_ref, b_ref, w_ref, o_ref):
    a = jnp.concatenate([acc_ref[0], acc_ref[1]], axis=1)
    h = jnp.concatenate([hp_ref[0], hp_ref[1]], axis=1)
    dis = dis_ref[...]
    x = jnp.maximum(dis * (a + h) + b_ref[...], 0.0)
    xs = dis * x
    o_ref[...] = jnp.dot(xs, w_ref[...], preferred_element_type=jnp.float32)


def _mid(dis, acc, hp, b, w, dh_out):
    dh = hp.shape[2]
    return pl.pallas_call(
        _mid_body,
        grid=(N // NODE_BLK,),
        in_specs=[
            pl.BlockSpec((NODE_BLK, 1), lambda i: (i, 0)),
            pl.BlockSpec((2, NODE_BLK, dh), lambda i: (0, i, 0)),
            pl.BlockSpec((2, NODE_BLK, dh), lambda i: (0, i, 0)),
            pl.BlockSpec((1, 2 * dh), lambda i: (0, 0)),
            pl.BlockSpec((2 * dh, dh_out), lambda i: (0, 0)),
        ],
        out_specs=pl.BlockSpec((NODE_BLK, dh_out), lambda i: (i, 0)),
        out_shape=jax.ShapeDtypeStruct((N, dh_out), jnp.float32),
    )(dis, acc, hp, b, w)


def _finish_body(dis_ref, acc_ref, hp_ref, b_ref, o_ref):
    a = acc_ref[0] + acc_ref[1]
    out = dis_ref[...] * (a + hp_ref[...]) + b_ref[...]
    nrm = jnp.sqrt(jnp.sum(out * out, axis=1, keepdims=True))
    o_ref[...] = out / jnp.maximum(nrm, 1e-12)


def _finish(dis, acc, hp, b):
    dh = hp.shape[1]
    return pl.pallas_call(
        _finish_body,
        grid=(N // NODE_BLK,),
        in_specs=[
            pl.BlockSpec((NODE_BLK, 1), lambda i: (i, 0)),
            pl.BlockSpec((2, NODE_BLK, dh), lambda i: (0, i, 0)),
            pl.BlockSpec((NODE_BLK, dh), lambda i: (i, 0)),
            pl.BlockSpec((1, dh), lambda i: (0, 0)),
        ],
        out_specs=pl.BlockSpec((NODE_BLK, dh), lambda i: (i, 0)),
        out_shape=jax.ShapeDtypeStruct((N, dh), jnp.float32),
    )(dis, acc, hp, b)


def kernel(edge_index, edge_weight, emb, W1, b1, W2, b2):
    row = edge_index[0]
    col = edge_index[1]
    pad = E_PAD - E
    padi = (jnp.arange(pad, dtype=jnp.int32) * 997) % N
    rowp = jnp.concatenate([row, padi]).reshape(CHUNKS, 128)
    colp = jnp.concatenate([col, padi]).reshape(CHUNKS, 128)
    ewp = jnp.concatenate(
        [edge_weight, jnp.zeros((pad,), jnp.float32)]).reshape(CHUNKS, 128)
    pidx = jnp.concatenate([
        jnp.stack([rowp, colp,
                   lax.bitcast_convert_type(ewp, jnp.int32)], axis=1),
        jnp.zeros((CHUNKS_T - CHUNKS, 3, 128), jnp.int32),
    ])

    t1 = _matmul_split(emb, W1, 32)
    dparts = _sc_deg(colp, ewp)
    deg = 1.0 + dparts[0, :N] + dparts[1, :N]
    dis = (deg ** -0.5)[:, None]

    hp1 = _scale_split(dis, t1)
    acc1 = _sc_agg(hp1, pidx, 32)
    hp2 = _mid(dis, acc1, hp1, b1[None, :], W2, 32)
    acc2 = _sc_agg(hp2, pidx, 32, edge_split=True)
    return _finish(dis, acc2, hp2, b2[None, :])


# speedup vs baseline: 1.0236x; 1.0236x over previous
import functools

import jax
import jax.numpy as jnp
from jax import lax
from jax.experimental import pallas as pl
from jax.experimental.pallas import tpu as pltpu
from jax.experimental.pallas import tpu_sc as plsc

N = 50000
E = 800000
N_PAD = 51200
STRIPE = N_PAD // 16
NCH = 400
CHUNKS = NCH * 16
CHUNKS_T = CHUNKS + 16
E_PAD = CHUNKS * 128
NODE_BLK = 5000

_SC_MESH = plsc.VectorSubcoreMesh(core_axis_name="c", subcore_axis_name="s")
_PIB = jax.lax.GatherScatterMode.PROMISE_IN_BOUNDS


def _deg_body(col_hbm, ew_hbm, out_hbm, dacc, zero_v, col_v, ew_v, sem):
    c = lax.axis_index("c")
    s = lax.axis_index("s")

    @pl.loop(0, 512 // 16)
    def _(i):
        zero_v[pl.ds(i * 16, 16)] = jnp.zeros((16,), jnp.float32)

    for t in range(STRIPE // 512):
        pltpu.sync_copy(zero_v, dacc.at[pl.ds(s * STRIPE + t * 512, 512)])
    rem = STRIPE % 512
    if rem:
        pltpu.sync_copy(zero_v.at[pl.ds(0, rem)],
                        dacc.at[pl.ds(s * STRIPE + STRIPE - rem, rem)])
    plsc.subcore_barrier()

    w = c * 16 + s

    @pl.loop(0, 50)
    def _(win):
        cb = w * 200 + win * 4
        pltpu.sync_copy(col_hbm.at[pl.ds(cb, 4)], col_v)
        pltpu.sync_copy(ew_hbm.at[pl.ds(cb, 4)], ew_v)
        cps = [
            pltpu.async_copy(ew_v.at[j], dacc.at[col_v.at[j]], sem, add=True)
            for j in range(4)
        ]
        for cp in cps:
            cp.wait()

    plsc.subcore_barrier()
    pltpu.sync_copy(dacc.at[pl.ds(s * STRIPE, STRIPE)],
                    out_hbm.at[c].at[pl.ds(s * STRIPE, STRIPE)])


def _sc_deg(col2d, ew2d):
    kfn = pl.kernel(
        _deg_body,
        out_type=jax.ShapeDtypeStruct((2, N_PAD), jnp.float32),
        mesh=_SC_MESH,
        scratch_types=[
            pltpu.VMEM_SHARED((N_PAD,), jnp.float32),
            pltpu.VMEM((512,), jnp.float32),
            pltpu.VMEM((4, 128), jnp.int32),
            pltpu.VMEM((4, 128), jnp.float32),
            pltpu.SemaphoreType.DMA,
        ],
    )
    return kfn(col2d, ew2d)


def _agg_body(tab_hbm, pidx_hbm, acc_hbm,
              aacc, idx_v, rows_v, gsem, ssem, isem, *, dh, edge_split):
    c = lax.axis_index("c")
    s = lax.axis_index("s")
    nk = dh // 16
    if edge_split:
        nch = CHUNKS // 32
        sbase = (c * 16 + s) * nch
        tref = tab_hbm
    else:
        nch = NCH
        sbase = s * nch
        tref = tab_hbm.at[c]

    def buf(b):
        return rows_v.at[pl.ds(b * 128, 128)]

    @pl.loop(0, 512)
    def _(i):
        for k in range(nk):
            rows_v[i, pl.ds(k * 16, 16)] = jnp.zeros((16,), jnp.float32)

    for t in range(STRIPE // 512):
        pltpu.sync_copy(rows_v,
                        aacc.at[pl.ds(s * STRIPE + t * 512, 512)])
    rem = STRIPE % 512
    if rem:
        pltpu.sync_copy(rows_v.at[pl.ds(0, rem)],
                        aacc.at[pl.ds(s * STRIPE + STRIPE - rem, rem)])
    plsc.subcore_barrier()

    def issue_gather(ip, b, chunk):
        del chunk
        return pltpu.async_copy(
            tref.at[idx_v.at[ip].at[b].at[0]], buf(b), gsem)

    def drain_gather(ip, b):
        pltpu.make_async_copy(
            tref.at[idx_v.at[ip].at[b].at[0]], buf(b), gsem).wait()

    def issue_scatter(ip, b):
        return pltpu.async_copy(
            buf(b), aacc.at[idx_v.at[ip].at[b].at[1]], ssem, add=True)

    def drain_scatter(ip, b):
        pltpu.make_async_copy(
            buf(b), aacc.at[idx_v.at[ip].at[b].at[1]], ssem).wait()

    def issue_idx(ip, chunk):
        return pltpu.async_copy(pidx_hbm.at[pl.ds(chunk, 4)],
                                idx_v.at[ip], isem)

    def drain_idx(ip, chunk):
        pltpu.make_async_copy(pidx_hbm.at[pl.ds(chunk, 4)],
                              idx_v.at[ip], isem).wait()

    def multiply(ip, b):
        @plsc.parallel_loop(0, 128, unroll=8)
        def _(e):
            g16 = (e // 16) * 16
            ewg = plsc.bitcast(idx_v[ip, b, 2, pl.ds(g16, 16)], jnp.float32)
            bc = lax.gather(
                ewg, jnp.broadcast_to((e % 16)[None, None], (16, 1)),
                lax.GatherDimensionNumbers(
                    offset_dims=(), collapsed_slice_dims=(0,),
                    start_index_map=(0,)),
                slice_sizes=(1,), mode=_PIB)
            r = b * 128 + e
            for k in range(nk):
                rows_v[r, pl.ds(k * 16, 16)] = (
                    rows_v[r, pl.ds(k * 16, 16)] * bc)

    def half(ip, cb_mine, cb_next):
        for b in range(4):
            drain_gather(ip, b)
            multiply(ip, b)
            issue_scatter(ip, b)
        drain_idx(1 - ip, cb_next)
        for b in range(4):
            drain_scatter(ip, b)
        for b in range(4):
            issue_gather(1 - ip, b, cb_next)
        issue_idx(ip, cb_next + 4)

    pltpu.sync_copy(pidx_hbm.at[pl.ds(sbase, 4)], idx_v.at[0])
    for b in range(4):
        issue_gather(0, b, sbase + b)
    issue_idx(1, sbase + 4)

    @pl.loop(0, nch // 8)
    def _(k):
        cb = sbase + k * 8
        half(0, cb, cb + 4)
        half(1, cb + 4, cb + 8)

    for b in range(4):
        drain_gather(0, b)
    drain_idx(1, sbase)

    plsc.subcore_barrier()
    pltpu.sync_copy(aacc.at[pl.ds(s * STRIPE, STRIPE)],
                    acc_hbm.at[c].at[pl.ds(s * STRIPE, STRIPE)])


def _sc_agg(tab, pidx, dh, edge_split=False):
    kfn = pl.kernel(
        functools.partial(_agg_body, dh=dh, edge_split=edge_split),
        out_type=jax.ShapeDtypeStruct((2, N_PAD, dh), jnp.float32),
        mesh=_SC_MESH,
        compiler_params=pltpu.CompilerParams(use_tc_tiling_on_sc=False,
                                             needs_layout_passes=False),
        scratch_types=[
            pltpu.VMEM_SHARED((N_PAD, dh), jnp.float32),
            pltpu.VMEM((2, 4, 3, 128), jnp.int32),
            pltpu.VMEM((512, dh), jnp.float32),
            pltpu.SemaphoreType.DMA,
            pltpu.SemaphoreType.DMA,
            pltpu.SemaphoreType.DMA,
        ],
    )
    return kfn(tab, pidx)


def _mm_body(dis_ref, x_ref, w_ref, o_ref):
    xs = dis_ref[...] * x_ref[...]
    o_ref[0] = jnp.dot(xs, w_ref[0], preferred_element_type=jnp.float32)
    o_ref[1] = jnp.dot(xs, w_ref[1], preferred_element_type=jnp.float32)


def _scaled_matmul_split(dis, x, w, dh):
    d_in = x.shape[1]
    w2 = jnp.stack([w[:, :dh], w[:, dh:]])
    return pl.pallas_call(
        _mm_body,
        grid=(N // NODE_BLK,),
        in_specs=[
            pl.BlockSpec((NODE_BLK, 1), lambda i: (i, 0)),
            pl.BlockSpec((NODE_BLK, d_in), lambda i: (i, 0)),
            pl.BlockSpec((2, d_in, dh), lambda i: (0, 0, 0)),
        ],
        out_specs=pl.BlockSpec((2, NODE_BLK, dh), lambda i: (0, i, 0)),
        out_shape=jax.ShapeDtypeStruct((2, N, dh), jnp.float32),
    )(dis, x, w2)


def _mid_body(dis_ref, acc_ref, hp_ref, b_ref, w_ref, o_ref):
    a = jnp.concatenate([acc_ref[0], acc_ref[1]], axis=1)
    h = jnp.concatenate([hp_ref[0], hp_ref[1]], axis=1)
    dis = dis_ref[...]
    x = jnp.maximum(dis * (a + h) + b_ref[...], 0.0)
    xs = dis * x
    o_ref[...] = jnp.dot(xs, w_ref[...], preferred_element_type=jnp.float32)


def _mid(dis, acc, hp, b, w, dh_out):
    dh = hp.shape[2]
    return pl.pallas_call(
        _mid_body,
        grid=(N // NODE_BLK,),
        in_specs=[
            pl.BlockSpec((NODE_BLK, 1), lambda i: (i, 0)),
            pl.BlockSpec((2, NODE_BLK, dh), lambda i: (0, i, 0)),
            pl.BlockSpec((2, NODE_BLK, dh), lambda i: (0, i, 0)),
            pl.BlockSpec((1, 2 * dh), lambda i: (0, 0)),
            pl.BlockSpec((2 * dh, dh_out), lambda i: (0, 0)),
        ],
        out_specs=pl.BlockSpec((NODE_BLK, dh_out), lambda i: (i, 0)),
        out_shape=jax.ShapeDtypeStruct((N, dh_out), jnp.float32),
    )(dis, acc, hp, b, w)


def _finish_body(dis_ref, acc_ref, hp_ref, b_ref, o_ref):
    a = acc_ref[0] + acc_ref[1]
    out = dis_ref[...] * (a + hp_ref[...]) + b_ref[...]
    nrm = jnp.sqrt(jnp.sum(out * out, axis=1, keepdims=True))
    o_ref[...] = out / jnp.maximum(nrm, 1e-12)


def _finish(dis, acc, hp, b):
    dh = hp.shape[1]
    return pl.pallas_call(
        _finish_body,
        grid=(N // NODE_BLK,),
        in_specs=[
            pl.BlockSpec((NODE_BLK, 1), lambda i: (i, 0)),
            pl.BlockSpec((2, NODE_BLK, dh), lambda i: (0, i, 0)),
            pl.BlockSpec((NODE_BLK, dh), lambda i: (i, 0)),
            pl.BlockSpec((1, dh), lambda i: (0, 0)),
        ],
        out_specs=pl.BlockSpec((NODE_BLK, dh), lambda i: (i, 0)),
        out_shape=jax.ShapeDtypeStruct((N, dh), jnp.float32),
    )(dis, acc, hp, b)


def kernel(edge_index, edge_weight, emb, W1, b1, W2, b2):
    row = edge_index[0]
    col = edge_index[1]
    pad = E_PAD - E
    padi = (jnp.arange(pad, dtype=jnp.int32) * 997) % N
    rowp = jnp.concatenate([row, padi]).reshape(CHUNKS, 128)
    colp = jnp.concatenate([col, padi]).reshape(CHUNKS, 128)
    ewp = jnp.concatenate(
        [edge_weight, jnp.zeros((pad,), jnp.float32)]).reshape(CHUNKS, 128)
    pidx = jnp.concatenate([
        jnp.stack([rowp, colp,
                   lax.bitcast_convert_type(ewp, jnp.int32)], axis=1),
        jnp.zeros((CHUNKS_T - CHUNKS, 3, 128), jnp.int32),
    ])

    dparts = _sc_deg(colp, ewp)
    deg = 1.0 + dparts[0, :N] + dparts[1, :N]
    dis = (deg ** -0.5)[:, None]

    hp1 = _scaled_matmul_split(dis, emb, W1, 32)
    acc1 = _sc_agg(hp1, pidx, 32)
    hp2 = _mid(dis, acc1, hp1, b1[None, :], W2, 32)
    acc2 = _sc_agg(hp2, pidx, 32, edge_split=True)
    return _finish(dis, acc2, hp2, b2[None, :])
